# Initial kernel scaffold; baseline (speedup 1.0000x reference)
import jax
import jax.numpy as jnp
from jax import lax
from jax.experimental import pallas as pl
from jax.experimental.pallas import tpu as pltpu
from jax.experimental.pallas import tpu_sc as plsc

N_CLS = 1024
SEQ_LEN = 77
CTX_DIM = 512
N_CTX = 8
SUFFIX = SEQ_LEN - 1 - N_CTX

_info = plsc.get_sparse_core_info()
_NC = _info.num_cores
_NS = _info.num_subcores
_NW = _NC * _NS
_CPW = N_CLS // _NW


def _body(tokens_hbm, table_hbm, ctx_hbm, out_hbm, idx_v, rows_v, ctx_v, gsem):
    wid = lax.axis_index("s") * _NC + lax.axis_index("c")
    pltpu.sync_copy(ctx_hbm, ctx_v)

    def step(i, carry):
        c = wid * _CPW + i
        pltpu.sync_copy(tokens_hbm.at[c], idx_v)
        pltpu.async_copy(table_hbm.at[idx_v], rows_v, gsem).wait()
        pltpu.sync_copy(rows_v.at[pl.ds(0, 1)], out_hbm.at[c, pl.ds(0, 1)])
        pltpu.sync_copy(ctx_v, out_hbm.at[c, pl.ds(1, N_CTX)])
        pltpu.sync_copy(rows_v.at[pl.ds(1 + N_CTX, SUFFIX)],
                        out_hbm.at[c, pl.ds(1 + N_CTX, SUFFIX)])
        return carry

    lax.fori_loop(0, _CPW, step, 0)


def kernel(tokens, table, ctx):
    f = pl.kernel(
        _body,
        out_type=jax.ShapeDtypeStruct((N_CLS, SEQ_LEN, CTX_DIM), jnp.float32),
        mesh=plsc.VectorSubcoreMesh(core_axis_name="c", subcore_axis_name="s"),
        scratch_types=[
            pltpu.VMEM((SEQ_LEN,), jnp.int32),
            pltpu.VMEM((SEQ_LEN, CTX_DIM), jnp.float32),
            pltpu.VMEM((N_CTX, CTX_DIM), jnp.float32),
            pltpu.SemaphoreType.DMA,
        ],
    )
    return f(tokens, table, ctx)


# speedup vs baseline: 1.3997x; 1.3997x over previous
import jax
import jax.numpy as jnp
from jax import lax
from jax.experimental import pallas as pl
from jax.experimental.pallas import tpu as pltpu
from jax.experimental.pallas import tpu_sc as plsc

N_CLS = 1024
SEQ_LEN = 77
CTX_DIM = 512
N_CTX = 8
SUFFIX = SEQ_LEN - 1 - N_CTX

_info = plsc.get_sparse_core_info()
_NC = _info.num_cores
_NS = _info.num_subcores
_NW = _NC * _NS
_CPW = N_CLS // _NW


def _body(tokens_hbm, table_hbm, ctx_hbm, out_hbm, idx_v, rows_v, ctx_v, gsem):
    wid = lax.axis_index("s") * _NC + lax.axis_index("c")
    pltpu.sync_copy(ctx_hbm, ctx_v)

    def step(i, carry):
        c = wid * _CPW + i
        pltpu.sync_copy(tokens_hbm.at[c], idx_v)
        pltpu.async_copy(table_hbm.at[idx_v], rows_v, gsem).wait()
        pltpu.sync_copy(rows_v.at[pl.ds(0, 1)], out_hbm.at[c, pl.ds(0, 1)])
        pltpu.sync_copy(ctx_v, out_hbm.at[c, pl.ds(1, N_CTX)])
        pltpu.sync_copy(rows_v.at[pl.ds(1 + N_CTX, SUFFIX)],
                        out_hbm.at[c, pl.ds(1 + N_CTX, SUFFIX)])
        return carry

    lax.fori_loop(0, _CPW, step, 0)


def kernel(tokens, table, ctx):
    f = pl.kernel(
        _body,
        out_type=jax.ShapeDtypeStruct((N_CLS, SEQ_LEN, CTX_DIM), jnp.float32),
        mesh=plsc.VectorSubcoreMesh(core_axis_name="c", subcore_axis_name="s"),
        compiler_params=pltpu.CompilerParams(use_tc_tiling_on_sc=False),
        scratch_types=[
            pltpu.VMEM((SEQ_LEN,), jnp.int32),
            pltpu.VMEM((SEQ_LEN, CTX_DIM), jnp.float32),
            pltpu.VMEM((N_CTX, CTX_DIM), jnp.float32),
            pltpu.SemaphoreType.DMA,
        ],
    )
    return f(tokens, table, ctx)
